# baseline (speedup 1.0000x reference)
import functools

import jax
import jax.numpy as jnp
from jax import lax
from jax.experimental import pallas as pl
from jax.experimental.pallas import tpu as pltpu
from jax.experimental.pallas import tpu_sc as plsc

_EPS = 0.1
_K = 39935
_MININT = -2147483648
_NW = 32
_STAT = 16
_SC_ROWS = 32
_TC_BLOCK = 8


def _sc_body(x_hbm, tgt_hbm, out_hbm, row_v, hist_v, tgt_v, stat_v, *,
             sc_rows, vocab):
    rows_per = sc_rows // _NW
    chunks = vocab // 16
    cid = lax.axis_index("c")
    sid = lax.axis_index("s")
    wid = sid * 2 + cid
    lane = lax.iota(jnp.int32, 16)
    ones = jnp.ones((16,), jnp.int32)
    zeros16 = jnp.zeros((16,), jnp.int32)

    pltpu.sync_copy(tgt_hbm, tgt_v)

    def usort(x):
        b = plsc.bitcast(x, jnp.int32)
        return jnp.where(b < 0, ~b, b ^ _MININT)

    def zero_hist():
        def zb(bb, c):
            plsc.store_scatter(hist_v, [bb * 16 + lane], zeros16)
            return c
        lax.fori_loop(0, 256, zb, 0)

    def hist_pass(l, prefix, mx):
        shift_b = jnp.int32(24 - 8 * l)
        shift_p = jnp.int32(32 - 8 * l)

        def body(c, mx):
            x = row_v[pl.ds(c * 16, 16)]
            ub = usort(x)
            bucket = lax.shift_right_logical(ub, shift_b) & jnp.int32(0xFF)
            idx = bucket * 16 + lane
            if l == 0:
                plsc.addupdate_scatter(hist_v, [idx], ones)
                mx = jnp.maximum(mx, x)
            else:
                active = lax.shift_right_logical(ub, shift_p) == prefix
                plsc.addupdate_scatter(hist_v, [idx], ones, mask=active)
            return mx
        return lax.fori_loop(0, chunks, body, mx, unroll=8)

    def walk(k_rem):
        def wb(i, carry):
            cum, b0, above = carry
            bb = 255 - i
            tot = jnp.sum(plsc.load_gather(hist_v, [bb * 16 + lane]))
            plsc.store_scatter(hist_v, [bb * 16 + lane], zeros16)
            newcum = cum + tot
            crossing = (cum < k_rem) & (newcum >= k_rem)
            b0 = jnp.where(crossing, bb, b0)
            above = jnp.where(crossing, cum, above)
            return newcum, b0, above
        _, b0, above = lax.fori_loop(0, 256, wb, (jnp.int32(0), jnp.int32(0),
                                                  jnp.int32(0)))
        return b0, above

    def row_body(r, c):
        row = wid * rows_per + r
        pltpu.sync_copy(x_hbm.at[row], row_v)
        t_splat = plsc.load_gather(tgt_v, [jnp.full((16,), row, jnp.int32)])

        prefix = jnp.int32(0)
        k_rem = jnp.int32(_K)
        mx = jnp.full((16,), -jnp.inf, jnp.float32)
        for l in range(4):
            mx = hist_pass(l, prefix, mx)
            b0, above = walk(k_rem)
            k_rem = k_rem - above
            prefix = (prefix << 8) | b0
        t_us = prefix
        cnt_gt = jnp.int32(_K) - k_rem
        m = jnp.max(mx)
        t_bias = t_us ^ _MININT

        def fin(c, carry):
            se, sgt, xt, yt, cnteq = carry
            x = row_v[pl.ds(c * 16, 16)]
            ub = usort(x)
            se = se + jnp.exp(x - m)
            gt = (ub ^ _MININT) > t_bias
            sgt = sgt + jnp.where(gt, x, 0.0)
            gi = c * 16 + lane
            tm = gi == t_splat
            xt = xt + jnp.where(tm, x, 0.0)
            yt = yt + jnp.where(tm, ub, 0)
            eq = (ub == t_us) & (gi < t_splat)
            cnteq = cnteq + jnp.where(eq, 1, 0)
            return se, sgt, xt, yt, cnteq

        z_f = jnp.zeros((16,), jnp.float32)
        se, sgt, xt, yt, cnteq = lax.fori_loop(
            0, chunks, fin, (z_f, z_f, z_f, zeros16, zeros16), unroll=8)

        se_s = jnp.sum(se)
        sgt_s = jnp.sum(sgt)
        xt_s = jnp.sum(xt)
        yt_s = jnp.sum(yt)
        cnteq_s = jnp.sum(cnteq)

        tus_v = jnp.full((16,), t_us, jnp.int32)
        ty_v = jnp.where(tus_v < 0, tus_v ^ _MININT, ~tus_v)
        tf_v = plsc.bitcast(ty_v, jnp.float32)

        incl = ((yt_s ^ _MININT) > t_bias) | \
            ((yt_s == t_us) & (cnt_gt + cnteq_s < _K))
        incl_f = jnp.where(incl, 1.0, 0.0).astype(jnp.float32)

        stat = jnp.where(lane == 0, m, 0.0)
        stat = jnp.where(lane == 1, se_s, stat)
        stat = jnp.where(lane == 2, sgt_s, stat)
        stat = jnp.where(lane == 3, cnt_gt.astype(jnp.float32), stat)
        stat = jnp.where(lane == 4, tf_v, stat)
        stat = jnp.where(lane == 5, xt_s, stat)
        stat = jnp.where(lane == 6, incl_f, stat)
        plsc.store_scatter(stat_v, [r * _STAT + lane], stat)
        return c

    zero_hist()
    lax.fori_loop(0, rows_per, row_body, 0)
    pltpu.sync_copy(stat_v, out_hbm.at[wid])


def _tc_body(tgt_ref, x_ref, out_ref, y_ref, *, rows, vocab, batch, row0):
    i = pl.program_id(0)

    n_sl = 8
    sl_step = ((vocab // n_sl) // 128) * 128
    sl_starts = [s * sl_step for s in range(n_sl)]
    slices = list(zip(sl_starts, sl_starts[1:] + [vocab]))

    m_p, lo_p, hi_p = [], [], []
    for s0, s1 in slices:
        xs = x_ref[:, pl.ds(s0, s1 - s0)]
        bs = jax.lax.bitcast_convert_type(xs, jnp.int32)
        ys = bs ^ jnp.where(bs < 0, jnp.int32(0x7FFFFFFF), jnp.int32(0))
        y_ref[:, pl.ds(s0, s1 - s0)] = ys
        m_p.append(jnp.max(xs, axis=1, keepdims=True))
        lo_p.append(jnp.min(ys, axis=1, keepdims=True))
        hi_p.append(jnp.max(ys, axis=1, keepdims=True))
    m = functools.reduce(jnp.maximum, m_p)
    lo = functools.reduce(jnp.minimum, lo_p)
    hi = functools.reduce(jnp.maximum, hi_p)

    se = None
    for s0, s1 in slices:
        xs = x_ref[:, pl.ds(s0, s1 - s0)]
        p = jnp.sum(jnp.exp(xs - m), axis=1, keepdims=True)
        se = p if se is None else se + p
    lse = m + jnp.log(se)

    def count_ge(mid):
        tot = None
        for s0, s1 in slices:
            c = jnp.sum(jnp.where(y_ref[:, pl.ds(s0, s1 - s0)] >= mid,
                                  1.0, 0.0), axis=1, keepdims=True)
            tot = c if tot is None else tot + c
        return tot

    def step(_, carry):
        lo, hi = carry
        mid = (lo >> 1) + (hi >> 1) + (lo & hi & 1) + ((lo ^ hi) & 1)
        ge = count_ge(mid) >= float(_K)
        lo = jnp.where(ge, mid, lo)
        hi = jnp.where(ge, hi, mid - 1)
        return lo, hi

    lo, hi = jax.lax.fori_loop(0, 32, step, (lo, hi), unroll=False)
    t_sort = lo
    t_val = jax.lax.bitcast_convert_type(
        t_sort ^ jnp.where(t_sort < 0, jnp.int32(0x7FFFFFFF), jnp.int32(0)),
        jnp.float32)

    tcol = jnp.stack([tgt_ref[row0 + i * rows + r] for r in range(rows)]
                     ).reshape(rows, 1)

    cnt_gt = s_gt = x_t = y_t = cnt_eq_lt = None
    for s0, s1 in slices:
        xs = x_ref[:, pl.ds(s0, s1 - s0)]
        ys = y_ref[:, pl.ds(s0, s1 - s0)]
        lane = jax.lax.broadcasted_iota(jnp.int32, (rows, s1 - s0), 1) + s0
        gt = ys > t_sort
        tmask = lane == tcol
        c1 = jnp.sum(jnp.where(gt, 1.0, 0.0), axis=1, keepdims=True)
        c2 = jnp.sum(jnp.where(gt, xs, 0.0), axis=1, keepdims=True)
        c3 = jnp.sum(jnp.where(tmask, xs, 0.0), axis=1, keepdims=True)
        c4 = jnp.sum(jnp.where(tmask, ys, 0), axis=1, keepdims=True)
        c5 = jnp.sum(jnp.where((ys == t_sort) & (lane < tcol), 1.0, 0.0),
                     axis=1, keepdims=True)
        if cnt_gt is None:
            cnt_gt, s_gt, x_t, y_t, cnt_eq_lt = c1, c2, c3, c4, c5
        else:
            cnt_gt += c1
            s_gt += c2
            x_t += c3
            y_t += c4
            cnt_eq_lt += c5
    incl = (y_t > t_sort) | ((y_t == t_sort) & (cnt_gt + cnt_eq_lt
                                                < float(_K)))

    s_topk = s_gt + (_K - cnt_gt) * t_val
    l_sum_topk = _K * lse - s_topk
    l_t = lse - x_t
    loss_rows = (_EPS / _K) * (l_sum_topk - jnp.where(incl, l_t, 0.0)) \
        + (1.0 - _EPS) * l_t

    part = jnp.sum(loss_rows) / batch

    @pl.when(i == 0)
    def _():
        out_ref[0, 0] = 0.0

    out_ref[0, 0] += part


def _epi_body(s_ref, part_ref, out_ref, *, batch):
    s = s_ref[...]
    m = s[:, 0:1]
    se = s[:, 1:2]
    sgt = s[:, 2:3]
    cnt_gt = s[:, 3:4]
    t_f = s[:, 4:5]
    x_t = s[:, 5:6]
    incl = s[:, 6:7]
    lse = m + jnp.log(se)
    s_topk = sgt + (_K - cnt_gt) * t_f
    l_sum_topk = _K * lse - s_topk
    l_t = lse - x_t
    loss = (_EPS / _K) * (l_sum_topk - incl * l_t) + (1.0 - _EPS) * l_t
    out_ref[0, 0] = jnp.sum(loss) / batch + part_ref[0, 0]


def kernel(inputs, targets):
    B, V = inputs.shape
    t32 = targets.astype(jnp.int32)
    sc_rows = _SC_ROWS
    tc_rows = B - sc_rows
    rows_per = sc_rows // _NW

    sc = pl.kernel(
        functools.partial(_sc_body, sc_rows=sc_rows, vocab=V),
        out_type=jax.ShapeDtypeStruct((_NW, rows_per * _STAT), jnp.float32),
        mesh=plsc.VectorSubcoreMesh(core_axis_name="c", subcore_axis_name="s"),
        compiler_params=pltpu.CompilerParams(needs_layout_passes=False),
        scratch_types=[
            pltpu.VMEM((V,), jnp.float32),
            pltpu.VMEM((4096,), jnp.int32),
            pltpu.VMEM((B,), jnp.int32),
            pltpu.VMEM((rows_per * _STAT,), jnp.float32),
        ],
    )
    stats = sc(inputs, t32).reshape(sc_rows, _STAT)

    blk0 = sc_rows // _TC_BLOCK
    tc_part = pl.pallas_call(
        functools.partial(_tc_body, rows=_TC_BLOCK, vocab=V, batch=float(B),
                          row0=sc_rows),
        grid_spec=pltpu.PrefetchScalarGridSpec(
            num_scalar_prefetch=1,
            grid=(tc_rows // _TC_BLOCK,),
            in_specs=[pl.BlockSpec((_TC_BLOCK, V),
                                   lambda i, t: (i + blk0, 0))],
            out_specs=pl.BlockSpec(memory_space=pltpu.SMEM),
            scratch_shapes=[pltpu.VMEM((_TC_BLOCK, V), jnp.int32)],
        ),
        out_shape=jax.ShapeDtypeStruct((1, 1), jnp.float32),
    )(t32, inputs)

    out = pl.pallas_call(
        functools.partial(_epi_body, batch=float(B)),
        out_shape=jax.ShapeDtypeStruct((1, 1), jnp.float32),
        in_specs=[pl.BlockSpec((sc_rows, _STAT), lambda: (0, 0)),
                  pl.BlockSpec(memory_space=pltpu.SMEM)],
        out_specs=pl.BlockSpec(memory_space=pltpu.SMEM),
    )(stats, tc_part)
    return out[0, 0]


# speedup vs baseline: 1.1863x; 1.1863x over previous
import functools

import jax
import jax.numpy as jnp
from jax import lax
from jax.experimental import pallas as pl
from jax.experimental.pallas import tpu as pltpu
from jax.experimental.pallas import tpu_sc as plsc

_EPS = 0.1
_K = 39935
_MININT = -2147483648
_NW = 32
_STAT = 16
_SC_ROWS = 32
_TC_BLOCK = 8


def _sc_body(x_hbm, tgt_hbm, out_hbm, row_v, hist_v, tgt_v, stat_v, *,
             sc_rows, vocab):
    rows_per = sc_rows // _NW
    chunks = vocab // 16
    cid = lax.axis_index("c")
    sid = lax.axis_index("s")
    wid = sid * 2 + cid
    lane = lax.iota(jnp.int32, 16)
    ones = jnp.ones((16,), jnp.int32)
    zeros16 = jnp.zeros((16,), jnp.int32)

    pltpu.sync_copy(tgt_hbm, tgt_v)

    def usort(x):
        b = plsc.bitcast(x, jnp.int32)
        return jnp.where(b < 0, ~b, b ^ _MININT)

    def zero_hist():
        @plsc.parallel_loop(0, 256, 1, unroll=4)
        def _(bb):
            plsc.store_scatter(hist_v, [bb * 16 + lane], zeros16)

    def hist_pass(l, prefix, mx):
        shift_b = jnp.int32(24 - 8 * l)
        shift_p = jnp.int32(32 - 8 * l)

        def body(c, mx):
            x = row_v[pl.ds(c * 16, 16)]
            ub = usort(x)
            bucket = lax.shift_right_logical(ub, shift_b) & jnp.int32(0xFF)
            idx = bucket * 16 + lane
            if l == 0:
                plsc.addupdate_scatter(hist_v, [idx], ones)
                mx = jnp.maximum(mx, x)
            else:
                active = lax.shift_right_logical(ub, shift_p) == prefix
                plsc.addupdate_scatter(hist_v, [idx], ones, mask=active)
            return mx
        return plsc.parallel_loop(0, chunks, 1, unroll=8, carry=mx)(body)

    def walk(k_rem):
        def wb(i, carry):
            cum, b0, above = carry
            bb = 255 - i
            tot = jnp.sum(plsc.load_gather(hist_v, [bb * 16 + lane]))
            plsc.store_scatter(hist_v, [bb * 16 + lane], zeros16)
            newcum = cum + tot
            crossing = (cum < k_rem) & (newcum >= k_rem)
            b0 = jnp.where(crossing, bb, b0)
            above = jnp.where(crossing, cum, above)
            return newcum, b0, above
        _, b0, above = lax.fori_loop(0, 256, wb, (jnp.int32(0), jnp.int32(0),
                                                  jnp.int32(0)))
        return b0, above

    def row_body(r, c):
        row = wid * rows_per + r
        pltpu.sync_copy(x_hbm.at[row], row_v)
        t_splat = plsc.load_gather(tgt_v, [jnp.full((16,), row, jnp.int32)])

        prefix = jnp.int32(0)
        k_rem = jnp.int32(_K)
        mx = jnp.full((16,), -jnp.inf, jnp.float32)
        for l in range(4):
            mx = hist_pass(l, prefix, mx)
            b0, above = walk(k_rem)
            k_rem = k_rem - above
            prefix = (prefix << 8) | b0
        t_us = prefix
        cnt_gt = jnp.int32(_K) - k_rem
        m = jnp.max(mx)
        t_bias = t_us ^ _MININT

        def fin(c, carry):
            se, sgt, xt, yt, cnteq = carry
            x = row_v[pl.ds(c * 16, 16)]
            ub = usort(x)
            se = se + jnp.exp(x - m)
            gt = (ub ^ _MININT) > t_bias
            sgt = sgt + jnp.where(gt, x, 0.0)
            gi = c * 16 + lane
            tm = gi == t_splat
            xt = xt + jnp.where(tm, x, 0.0)
            yt = yt + jnp.where(tm, ub, 0)
            eq = (ub == t_us) & (gi < t_splat)
            cnteq = cnteq + jnp.where(eq, 1, 0)
            return se, sgt, xt, yt, cnteq

        z_f = jnp.zeros((16,), jnp.float32)
        se, sgt, xt, yt, cnteq = plsc.parallel_loop(
            0, chunks, 1, unroll=8,
            carry=(z_f, z_f, z_f, zeros16, zeros16))(fin)

        se_s = jnp.sum(se)
        sgt_s = jnp.sum(sgt)
        xt_s = jnp.sum(xt)
        yt_s = jnp.sum(yt)
        cnteq_s = jnp.sum(cnteq)

        tus_v = jnp.full((16,), t_us, jnp.int32)
        ty_v = jnp.where(tus_v < 0, tus_v ^ _MININT, ~tus_v)
        tf_v = plsc.bitcast(ty_v, jnp.float32)

        incl = ((yt_s ^ _MININT) > t_bias) | \
            ((yt_s == t_us) & (cnt_gt + cnteq_s < _K))
        incl_f = jnp.where(incl, 1.0, 0.0).astype(jnp.float32)

        stat = jnp.where(lane == 0, m, 0.0)
        stat = jnp.where(lane == 1, se_s, stat)
        stat = jnp.where(lane == 2, sgt_s, stat)
        stat = jnp.where(lane == 3, cnt_gt.astype(jnp.float32), stat)
        stat = jnp.where(lane == 4, tf_v, stat)
        stat = jnp.where(lane == 5, xt_s, stat)
        stat = jnp.where(lane == 6, incl_f, stat)
        plsc.store_scatter(stat_v, [r * _STAT + lane], stat)
        return c

    zero_hist()
    lax.fori_loop(0, rows_per, row_body, 0)
    pltpu.sync_copy(stat_v, out_hbm.at[wid])


def _tc_body(tgt_ref, x_ref, out_ref, y_ref, *, rows, vocab, batch, row0):
    i = pl.program_id(0)

    n_sl = 8
    sl_step = ((vocab // n_sl) // 128) * 128
    sl_starts = [s * sl_step for s in range(n_sl)]
    slices = list(zip(sl_starts, sl_starts[1:] + [vocab]))

    m_p, lo_p, hi_p = [], [], []
    for s0, s1 in slices:
        xs = x_ref[:, pl.ds(s0, s1 - s0)]
        bs = jax.lax.bitcast_convert_type(xs, jnp.int32)
        ys = bs ^ jnp.where(bs < 0, jnp.int32(0x7FFFFFFF), jnp.int32(0))
        y_ref[:, pl.ds(s0, s1 - s0)] = ys
        m_p.append(jnp.max(xs, axis=1, keepdims=True))
        lo_p.append(jnp.min(ys, axis=1, keepdims=True))
        hi_p.append(jnp.max(ys, axis=1, keepdims=True))
    m = functools.reduce(jnp.maximum, m_p)
    lo = functools.reduce(jnp.minimum, lo_p)
    hi = functools.reduce(jnp.maximum, hi_p)

    se = None
    for s0, s1 in slices:
        xs = x_ref[:, pl.ds(s0, s1 - s0)]
        p = jnp.sum(jnp.exp(xs - m), axis=1, keepdims=True)
        se = p if se is None else se + p
    lse = m + jnp.log(se)

    def count_ge(mid):
        tot = None
        for s0, s1 in slices:
            c = jnp.sum(jnp.where(y_ref[:, pl.ds(s0, s1 - s0)] >= mid,
                                  1.0, 0.0), axis=1, keepdims=True)
            tot = c if tot is None else tot + c
        return tot

    def step(_, carry):
        lo, hi = carry
        mid = (lo >> 1) + (hi >> 1) + (lo & hi & 1) + ((lo ^ hi) & 1)
        ge = count_ge(mid) >= float(_K)
        lo = jnp.where(ge, mid, lo)
        hi = jnp.where(ge, hi, mid - 1)
        return lo, hi

    lo, hi = jax.lax.fori_loop(0, 32, step, (lo, hi), unroll=False)
    t_sort = lo
    t_val = jax.lax.bitcast_convert_type(
        t_sort ^ jnp.where(t_sort < 0, jnp.int32(0x7FFFFFFF), jnp.int32(0)),
        jnp.float32)

    tcol = jnp.stack([tgt_ref[row0 + i * rows + r] for r in range(rows)]
                     ).reshape(rows, 1)

    cnt_gt = s_gt = x_t = y_t = cnt_eq_lt = None
    for s0, s1 in slices:
        xs = x_ref[:, pl.ds(s0, s1 - s0)]
        ys = y_ref[:, pl.ds(s0, s1 - s0)]
        lane = jax.lax.broadcasted_iota(jnp.int32, (rows, s1 - s0), 1) + s0
        gt = ys > t_sort
        tmask = lane == tcol
        c1 = jnp.sum(jnp.where(gt, 1.0, 0.0), axis=1, keepdims=True)
        c2 = jnp.sum(jnp.where(gt, xs, 0.0), axis=1, keepdims=True)
        c3 = jnp.sum(jnp.where(tmask, xs, 0.0), axis=1, keepdims=True)
        c4 = jnp.sum(jnp.where(tmask, ys, 0), axis=1, keepdims=True)
        c5 = jnp.sum(jnp.where((ys == t_sort) & (lane < tcol), 1.0, 0.0),
                     axis=1, keepdims=True)
        if cnt_gt is None:
            cnt_gt, s_gt, x_t, y_t, cnt_eq_lt = c1, c2, c3, c4, c5
        else:
            cnt_gt += c1
            s_gt += c2
            x_t += c3
            y_t += c4
            cnt_eq_lt += c5
    incl = (y_t > t_sort) | ((y_t == t_sort) & (cnt_gt + cnt_eq_lt
                                                < float(_K)))

    s_topk = s_gt + (_K - cnt_gt) * t_val
    l_sum_topk = _K * lse - s_topk
    l_t = lse - x_t
    loss_rows = (_EPS / _K) * (l_sum_topk - jnp.where(incl, l_t, 0.0)) \
        + (1.0 - _EPS) * l_t

    part = jnp.sum(loss_rows) / batch

    @pl.when(i == 0)
    def _():
        out_ref[0, 0] = 0.0

    out_ref[0, 0] += part


def _epi_body(s_ref, part_ref, out_ref, *, batch):
    s = s_ref[...]
    m = s[:, 0:1]
    se = s[:, 1:2]
    sgt = s[:, 2:3]
    cnt_gt = s[:, 3:4]
    t_f = s[:, 4:5]
    x_t = s[:, 5:6]
    incl = s[:, 6:7]
    lse = m + jnp.log(se)
    s_topk = sgt + (_K - cnt_gt) * t_f
    l_sum_topk = _K * lse - s_topk
    l_t = lse - x_t
    loss = (_EPS / _K) * (l_sum_topk - incl * l_t) + (1.0 - _EPS) * l_t
    out_ref[0, 0] = jnp.sum(loss) / batch + part_ref[0, 0]


def kernel(inputs, targets):
    B, V = inputs.shape
    t32 = targets.astype(jnp.int32)
    sc_rows = _SC_ROWS
    tc_rows = B - sc_rows
    rows_per = sc_rows // _NW

    sc = pl.kernel(
        functools.partial(_sc_body, sc_rows=sc_rows, vocab=V),
        out_type=jax.ShapeDtypeStruct((_NW, rows_per * _STAT), jnp.float32),
        mesh=plsc.VectorSubcoreMesh(core_axis_name="c", subcore_axis_name="s"),
        compiler_params=pltpu.CompilerParams(needs_layout_passes=False),
        scratch_types=[
            pltpu.VMEM((V,), jnp.float32),
            pltpu.VMEM((4096,), jnp.int32),
            pltpu.VMEM((B,), jnp.int32),
            pltpu.VMEM((rows_per * _STAT,), jnp.float32),
        ],
    )
    stats = sc(inputs, t32).reshape(sc_rows, _STAT)

    blk0 = sc_rows // _TC_BLOCK
    tc_part = pl.pallas_call(
        functools.partial(_tc_body, rows=_TC_BLOCK, vocab=V, batch=float(B),
                          row0=sc_rows),
        grid_spec=pltpu.PrefetchScalarGridSpec(
            num_scalar_prefetch=1,
            grid=(tc_rows // _TC_BLOCK,),
            in_specs=[pl.BlockSpec((_TC_BLOCK, V),
                                   lambda i, t: (i + blk0, 0))],
            out_specs=pl.BlockSpec(memory_space=pltpu.SMEM),
            scratch_shapes=[pltpu.VMEM((_TC_BLOCK, V), jnp.int32)],
        ),
        out_shape=jax.ShapeDtypeStruct((1, 1), jnp.float32),
    )(t32, inputs)

    out = pl.pallas_call(
        functools.partial(_epi_body, batch=float(B)),
        out_shape=jax.ShapeDtypeStruct((1, 1), jnp.float32),
        in_specs=[pl.BlockSpec((sc_rows, _STAT), lambda: (0, 0)),
                  pl.BlockSpec(memory_space=pltpu.SMEM)],
        out_specs=pl.BlockSpec(memory_space=pltpu.SMEM),
    )(stats, tc_part)
    return out[0, 0]


# speedup vs baseline: 1.4662x; 1.2359x over previous
import functools

import jax
import jax.numpy as jnp
from jax import lax
from jax.experimental import pallas as pl
from jax.experimental.pallas import tpu as pltpu
from jax.experimental.pallas import tpu_sc as plsc

_EPS = 0.1
_K = 39935
_MININT = -2147483648
_NW = 32
_STAT = 16
_SC_ROWS = 64
_TC_BLOCK = 8


def _sc_body(x_hbm, tgt_hbm, out_hbm, row_v, hist_v, tgt_v, stat_v, *,
             sc_rows, vocab):
    rows_per = sc_rows // _NW
    chunks = vocab // 16
    cid = lax.axis_index("c")
    sid = lax.axis_index("s")
    wid = sid * 2 + cid
    lane = lax.iota(jnp.int32, 16)
    ones = jnp.ones((16,), jnp.int32)
    zeros16 = jnp.zeros((16,), jnp.int32)

    pltpu.sync_copy(tgt_hbm, tgt_v)

    def usort(x):
        b = plsc.bitcast(x, jnp.int32)
        return jnp.where(b < 0, ~b, b ^ _MININT)

    def zero_hist():
        @plsc.parallel_loop(0, 256, 1, unroll=4)
        def _(bb):
            plsc.store_scatter(hist_v, [bb * 16 + lane], zeros16)

    def hist_pass(l, prefix, mx):
        shift_b = jnp.int32(24 - 8 * l)
        shift_p = jnp.int32(32 - 8 * l)

        def body(c, mx):
            x = row_v[pl.ds(c * 16, 16)]
            ub = usort(x)
            bucket = lax.shift_right_logical(ub, shift_b) & jnp.int32(0xFF)
            idx = bucket * 16 + lane
            if l == 0:
                plsc.addupdate_scatter(hist_v, [idx], ones)
                mx = jnp.maximum(mx, x)
            else:
                active = lax.shift_right_logical(ub, shift_p) == prefix
                plsc.addupdate_scatter(hist_v, [idx], ones, mask=active)
            return mx
        return plsc.parallel_loop(0, chunks, 1, unroll=8, carry=mx)(body)

    def walk(k_rem):
        def wb(i, carry):
            cum, b0, above = carry
            bb = 255 - i
            tot = jnp.sum(plsc.load_gather(hist_v, [bb * 16 + lane]))
            plsc.store_scatter(hist_v, [bb * 16 + lane], zeros16)
            newcum = cum + tot
            crossing = (cum < k_rem) & (newcum >= k_rem)
            b0 = jnp.where(crossing, bb, b0)
            above = jnp.where(crossing, cum, above)
            return newcum, b0, above
        _, b0, above = lax.fori_loop(0, 256, wb, (jnp.int32(0), jnp.int32(0),
                                                  jnp.int32(0)))
        return b0, above

    def row_body(r, c):
        row = wid * rows_per + r
        pltpu.sync_copy(x_hbm.at[row], row_v)
        t_splat = plsc.load_gather(tgt_v, [jnp.full((16,), row, jnp.int32)])

        prefix = jnp.int32(0)
        k_rem = jnp.int32(_K)
        mx = jnp.full((16,), -jnp.inf, jnp.float32)
        for l in range(4):
            mx = hist_pass(l, prefix, mx)
            b0, above = walk(k_rem)
            k_rem = k_rem - above
            prefix = (prefix << 8) | b0
        t_us = prefix
        cnt_gt = jnp.int32(_K) - k_rem
        m = jnp.max(mx)
        t_bias = t_us ^ _MININT

        def fin(c, carry):
            se, sgt, xt, yt, cnteq = carry
            x = row_v[pl.ds(c * 16, 16)]
            ub = usort(x)
            se = se + jnp.exp(x - m)
            gt = (ub ^ _MININT) > t_bias
            sgt = sgt + jnp.where(gt, x, 0.0)
            gi = c * 16 + lane
            tm = gi == t_splat
            xt = xt + jnp.where(tm, x, 0.0)
            yt = yt + jnp.where(tm, ub, 0)
            eq = (ub == t_us) & (gi < t_splat)
            cnteq = cnteq + jnp.where(eq, 1, 0)
            return se, sgt, xt, yt, cnteq

        z_f = jnp.zeros((16,), jnp.float32)
        se, sgt, xt, yt, cnteq = plsc.parallel_loop(
            0, chunks, 1, unroll=8,
            carry=(z_f, z_f, z_f, zeros16, zeros16))(fin)

        se_s = jnp.sum(se)
        sgt_s = jnp.sum(sgt)
        xt_s = jnp.sum(xt)
        yt_s = jnp.sum(yt)
        cnteq_s = jnp.sum(cnteq)

        tus_v = jnp.full((16,), t_us, jnp.int32)
        ty_v = jnp.where(tus_v < 0, tus_v ^ _MININT, ~tus_v)
        tf_v = plsc.bitcast(ty_v, jnp.float32)

        incl = ((yt_s ^ _MININT) > t_bias) | \
            ((yt_s == t_us) & (cnt_gt + cnteq_s < _K))
        incl_f = jnp.where(incl, 1.0, 0.0).astype(jnp.float32)

        stat = jnp.where(lane == 0, m, 0.0)
        stat = jnp.where(lane == 1, se_s, stat)
        stat = jnp.where(lane == 2, sgt_s, stat)
        stat = jnp.where(lane == 3, cnt_gt.astype(jnp.float32), stat)
        stat = jnp.where(lane == 4, tf_v, stat)
        stat = jnp.where(lane == 5, xt_s, stat)
        stat = jnp.where(lane == 6, incl_f, stat)
        plsc.store_scatter(stat_v, [r * _STAT + lane], stat)
        return c

    zero_hist()
    lax.fori_loop(0, rows_per, row_body, 0)
    pltpu.sync_copy(stat_v, out_hbm.at[wid])


def _tc_body(tgt_ref, x_ref, out_ref, y_ref, *, rows, vocab, batch, row0):
    i = pl.program_id(0)

    n_sl = 8
    sl_step = ((vocab // n_sl) // 128) * 128
    sl_starts = [s * sl_step for s in range(n_sl)]
    slices = list(zip(sl_starts, sl_starts[1:] + [vocab]))

    m_p, lo_p, hi_p = [], [], []
    for s0, s1 in slices:
        xs = x_ref[:, pl.ds(s0, s1 - s0)]
        bs = jax.lax.bitcast_convert_type(xs, jnp.int32)
        ys = bs ^ jnp.where(bs < 0, jnp.int32(0x7FFFFFFF), jnp.int32(0))
        y_ref[:, pl.ds(s0, s1 - s0)] = ys
        m_p.append(jnp.max(xs, axis=1, keepdims=True))
        lo_p.append(jnp.min(ys, axis=1, keepdims=True))
        hi_p.append(jnp.max(ys, axis=1, keepdims=True))
    m = functools.reduce(jnp.maximum, m_p)
    lo = functools.reduce(jnp.minimum, lo_p)
    hi = functools.reduce(jnp.maximum, hi_p)

    se = None
    for s0, s1 in slices:
        xs = x_ref[:, pl.ds(s0, s1 - s0)]
        p = jnp.sum(jnp.exp(xs - m), axis=1, keepdims=True)
        se = p if se is None else se + p
    lse = m + jnp.log(se)

    def count_ge(mid):
        tot = None
        for s0, s1 in slices:
            c = jnp.sum(jnp.where(y_ref[:, pl.ds(s0, s1 - s0)] >= mid,
                                  1.0, 0.0), axis=1, keepdims=True)
            tot = c if tot is None else tot + c
        return tot

    def step(_, carry):
        lo, hi = carry
        mid = (lo >> 1) + (hi >> 1) + (lo & hi & 1) + ((lo ^ hi) & 1)
        ge = count_ge(mid) >= float(_K)
        lo = jnp.where(ge, mid, lo)
        hi = jnp.where(ge, hi, mid - 1)
        return lo, hi

    lo, hi = jax.lax.fori_loop(0, 32, step, (lo, hi), unroll=False)
    t_sort = lo
    t_val = jax.lax.bitcast_convert_type(
        t_sort ^ jnp.where(t_sort < 0, jnp.int32(0x7FFFFFFF), jnp.int32(0)),
        jnp.float32)

    tcol = jnp.stack([tgt_ref[row0 + i * rows + r] for r in range(rows)]
                     ).reshape(rows, 1)

    cnt_gt = s_gt = x_t = y_t = cnt_eq_lt = None
    for s0, s1 in slices:
        xs = x_ref[:, pl.ds(s0, s1 - s0)]
        ys = y_ref[:, pl.ds(s0, s1 - s0)]
        lane = jax.lax.broadcasted_iota(jnp.int32, (rows, s1 - s0), 1) + s0
        gt = ys > t_sort
        tmask = lane == tcol
        c1 = jnp.sum(jnp.where(gt, 1.0, 0.0), axis=1, keepdims=True)
        c2 = jnp.sum(jnp.where(gt, xs, 0.0), axis=1, keepdims=True)
        c3 = jnp.sum(jnp.where(tmask, xs, 0.0), axis=1, keepdims=True)
        c4 = jnp.sum(jnp.where(tmask, ys, 0), axis=1, keepdims=True)
        c5 = jnp.sum(jnp.where((ys == t_sort) & (lane < tcol), 1.0, 0.0),
                     axis=1, keepdims=True)
        if cnt_gt is None:
            cnt_gt, s_gt, x_t, y_t, cnt_eq_lt = c1, c2, c3, c4, c5
        else:
            cnt_gt += c1
            s_gt += c2
            x_t += c3
            y_t += c4
            cnt_eq_lt += c5
    incl = (y_t > t_sort) | ((y_t == t_sort) & (cnt_gt + cnt_eq_lt
                                                < float(_K)))

    s_topk = s_gt + (_K - cnt_gt) * t_val
    l_sum_topk = _K * lse - s_topk
    l_t = lse - x_t
    loss_rows = (_EPS / _K) * (l_sum_topk - jnp.where(incl, l_t, 0.0)) \
        + (1.0 - _EPS) * l_t

    part = jnp.sum(loss_rows) / batch

    @pl.when(i == 0)
    def _():
        out_ref[0, 0] = 0.0

    out_ref[0, 0] += part


def _epi_body(s_ref, part_ref, out_ref, *, batch):
    s = s_ref[...]
    m = s[:, 0:1]
    se = s[:, 1:2]
    sgt = s[:, 2:3]
    cnt_gt = s[:, 3:4]
    t_f = s[:, 4:5]
    x_t = s[:, 5:6]
    incl = s[:, 6:7]
    lse = m + jnp.log(se)
    s_topk = sgt + (_K - cnt_gt) * t_f
    l_sum_topk = _K * lse - s_topk
    l_t = lse - x_t
    loss = (_EPS / _K) * (l_sum_topk - incl * l_t) + (1.0 - _EPS) * l_t
    out_ref[0, 0] = jnp.sum(loss) / batch + part_ref[0, 0]


def kernel(inputs, targets):
    B, V = inputs.shape
    t32 = targets.astype(jnp.int32)
    sc_rows = _SC_ROWS
    tc_rows = B - sc_rows
    rows_per = sc_rows // _NW

    sc = pl.kernel(
        functools.partial(_sc_body, sc_rows=sc_rows, vocab=V),
        out_type=jax.ShapeDtypeStruct((_NW, rows_per * _STAT), jnp.float32),
        mesh=plsc.VectorSubcoreMesh(core_axis_name="c", subcore_axis_name="s"),
        compiler_params=pltpu.CompilerParams(needs_layout_passes=False),
        scratch_types=[
            pltpu.VMEM((V,), jnp.float32),
            pltpu.VMEM((4096,), jnp.int32),
            pltpu.VMEM((B,), jnp.int32),
            pltpu.VMEM((rows_per * _STAT,), jnp.float32),
        ],
    )
    stats = sc(inputs, t32).reshape(sc_rows, _STAT)

    blk0 = sc_rows // _TC_BLOCK
    tc_part = pl.pallas_call(
        functools.partial(_tc_body, rows=_TC_BLOCK, vocab=V, batch=float(B),
                          row0=sc_rows),
        grid_spec=pltpu.PrefetchScalarGridSpec(
            num_scalar_prefetch=1,
            grid=(tc_rows // _TC_BLOCK,),
            in_specs=[pl.BlockSpec((_TC_BLOCK, V),
                                   lambda i, t: (i + blk0, 0))],
            out_specs=pl.BlockSpec(memory_space=pltpu.SMEM),
            scratch_shapes=[pltpu.VMEM((_TC_BLOCK, V), jnp.int32)],
        ),
        out_shape=jax.ShapeDtypeStruct((1, 1), jnp.float32),
    )(t32, inputs)

    out = pl.pallas_call(
        functools.partial(_epi_body, batch=float(B)),
        out_shape=jax.ShapeDtypeStruct((1, 1), jnp.float32),
        in_specs=[pl.BlockSpec((sc_rows, _STAT), lambda: (0, 0)),
                  pl.BlockSpec(memory_space=pltpu.SMEM)],
        out_specs=pl.BlockSpec(memory_space=pltpu.SMEM),
    )(stats, tc_part)
    return out[0, 0]
